# baseline (speedup 1.0000x reference)
import functools

import jax
import jax.numpy as jnp
from jax import lax
from jax.experimental import pallas as pl
from jax.experimental.pallas import tpu as pltpu
from jax.experimental.pallas import tpu_sc as plsc

N = 10000
E = 160000
H = 32
O = 10000

NC = 2
NS = 16
NW = NC * NS
EPT = E // NW
CH = 1000
NPT = N // NS
BR = 200
NB = N // BR

_mesh = plsc.VectorSubcoreMesh(core_axis_name="c", subcore_axis_name="s")

_Z16 = functools.partial(jnp.zeros, (16,), jnp.float32)


def _lane_bcast(vec16, u):
    return lax.gather(
        vec16,
        jnp.full((16, 1), u, jnp.int32),
        lax.GatherDimensionNumbers(
            offset_dims=(), collapsed_slice_dims=(0,), start_index_map=(0,)
        ),
        (1,),
        mode=lax.GatherScatterMode.PROMISE_IN_BOUNDS,
    )


def _vreg_loop(total, body16):
    nfull = total // 16

    def fbody(i, _):
        body16(i * 16, None)
        return 0

    lax.fori_loop(0, nfull, fbody, 0)
    rem = total - nfull * 16
    if rem:
        lane = lax.iota(jnp.int32, 16)
        body16(total - 16, lane >= (16 - rem))


def _deg_body(dst_h, src_h, ew_h, cn_h, outw_h, outc_h, outm_h,
              dst_v, src_v, ew_v, cn_v, accw, accc, acch):
    cid = lax.axis_index("c")
    sid = lax.axis_index("s")
    wid = cid * NS + sid
    z = _Z16()

    def zbody(i, _):
        accw[pl.ds(i * 16, 16)] = z
        accc[pl.ds(i * 16, 16)] = z
        acch[pl.ds(i * 16, 16)] = z
        return 0

    lax.fori_loop(0, N // 16, zbody, 0)

    base = wid * EPT
    pltpu.sync_copy(dst_h.at[pl.ds(base, EPT)], dst_v)
    pltpu.sync_copy(src_h.at[pl.ds(base, EPT)], src_v)
    pltpu.sync_copy(ew_h.at[pl.ds(base, EPT)], ew_v)
    pltpu.sync_copy(cn_h, cn_v)

    one = jnp.ones((16,), jnp.float32)
    cn16 = cn_v[pl.ds(0, 16)]

    def body16(off, mask):
        d16 = dst_v[pl.ds(off, 16)]
        hit16 = jnp.where(src_v[pl.ds(off, 16)] == cn16, 1.0, 0.0)
        plsc.addupdate_scatter(accw, [d16], ew_v[pl.ds(off, 16)], mask=mask)
        plsc.addupdate_scatter(accc, [d16], one, mask=mask)
        plsc.addupdate_scatter(acch, [d16], hit16, mask=mask)

    _vreg_loop(EPT, body16)

    pltpu.sync_copy(accw, outw_h.at[wid, 0])
    pltpu.sync_copy(accc, outc_h.at[wid, 0])
    pltpu.sync_copy(acch, outm_h.at[wid, 0])


_deg_call = pl.kernel(
    _deg_body,
    out_type=[
        jax.ShapeDtypeStruct((NW, 1, N), jnp.float32),
        jax.ShapeDtypeStruct((NW, 1, N), jnp.float32),
        jax.ShapeDtypeStruct((NW, 1, N), jnp.float32),
    ],
    mesh=_mesh,
    scratch_types=[
        pltpu.VMEM((EPT,), jnp.int32),
        pltpu.VMEM((EPT,), jnp.int32),
        pltpu.VMEM((EPT,), jnp.float32),
        pltpu.VMEM((16,), jnp.int32),
        pltpu.VMEM((N,), jnp.float32),
        pltpu.VMEM((N,), jnp.float32),
        pltpu.VMEM((N,), jnp.float32),
    ],
    compiler_params=pltpu.CompilerParams(needs_layout_passes=False, use_tc_tiling_on_sc=False),
)


def _agg_body(scale, *refs):
    if scale:
        (y_h, src_h, dst_h, w_h, d1_h, out_h,
         src_v, dst_v, w_v, d1_v, rows0, rows1, acc_sh, sem0, sem1) = refs
    else:
        (y_h, src_h, dst_h, out_h,
         src_v, dst_v, rows0, rows1, acc_sh, sem0, sem1) = refs
    cid = lax.axis_index("c")
    sid = lax.axis_index("s")
    wid = cid * NS + sid
    z = _Z16()

    def zbody(i, _):
        rows0[i, pl.ds(0, 16)] = z
        rows0[i, pl.ds(16, 16)] = z
        return 0

    lax.fori_loop(0, 640, zbody, 0)

    @pl.when(sid < NS - 1)
    def _zfill():
        pltpu.sync_copy(rows0.at[pl.ds(0, 624)], acc_sh.at[pl.ds(sid * 624, 624)])

    @pl.when(sid == NS - 1)
    def _zfill_last():
        pltpu.sync_copy(rows0.at[pl.ds(0, 640)], acc_sh.at[pl.ds(9360, 640)])

    plsc.subcore_barrier()

    base = wid * EPT
    pltpu.sync_copy(src_h.at[pl.ds(base, EPT)], src_v.at[pl.ds(0, EPT)])
    pltpu.sync_copy(dst_h.at[wid], dst_v)
    if scale:
        src_v[pl.ds(EPT, 16)] = jnp.zeros((16,), jnp.int32)
        pltpu.sync_copy(w_h.at[pl.ds(base, EPT)], w_v.at[pl.ds(0, EPT)])
        pltpu.sync_copy(d1_h, d1_v)

    rows = (rows0, rows1)
    sems = (sem0, sem1)
    NCH = EPT // CH
    descs = [None] * NCH
    descs[0] = pltpu.async_copy(y_h.at[src_v.at[pl.ds(0, CH)]], rows0, sems[0])
    for c in range(NCH):
        b = c % 2
        descs[c].wait()
        if c + 1 < NCH:
            descs[c + 1] = pltpu.async_copy(
                y_h.at[src_v.at[pl.ds((c + 1) * CH, CH)]],
                rows[(c + 1) % 2],
                sems[(c + 1) % 2],
            )
        if scale:
            rv = rows[b]
            z16i = jnp.zeros((16,), jnp.int32)

            def sbody(i, _):
                off = c * CH + i * 8
                s16 = src_v[pl.ds(off, 16)]
                w16 = w_v[pl.ds(off, 16)] * plsc.load_gather(d1_v, [s16])
                for u in range(8):
                    e = i * 8 + u
                    wb = _lane_bcast(w16, u)
                    rv[e, pl.ds(0, 16)] = rv[e, pl.ds(0, 16)] * wb
                    rv[e, pl.ds(16, 16)] = rv[e, pl.ds(16, 16)] * wb
                return 0

            lax.fori_loop(0, CH // 8, sbody, 0)
        pltpu.sync_copy(rows[b], acc_sh.at[dst_v.at[c]], add=True)

    plsc.subcore_barrier()

    @pl.when(sid < NS - 1)
    def _wout():
        pltpu.sync_copy(
            acc_sh.at[pl.ds(sid * 624, 624)],
            out_h.at[cid, pl.ds(sid * 624, 624)],
        )

    @pl.when(sid == NS - 1)
    def _wout_last():
        pltpu.sync_copy(
            acc_sh.at[pl.ds(9360, 640)],
            out_h.at[cid, pl.ds(9360, 640)],
        )


def _make_agg(scale):
    if scale:
        scratch = [
            pltpu.VMEM((EPT + 16,), jnp.int32),
            pltpu.VMEM((EPT // CH, CH), jnp.int32),
            pltpu.VMEM((EPT + 16,), jnp.float32),
            pltpu.VMEM((N,), jnp.float32),
        ]
    else:
        scratch = [
            pltpu.VMEM((EPT,), jnp.int32),
            pltpu.VMEM((EPT // CH, CH), jnp.int32),
        ]
    scratch += [
        pltpu.VMEM((CH, H), jnp.float32),
        pltpu.VMEM((CH, H), jnp.float32),
        pltpu.VMEM_SHARED((N, H), jnp.float32),
        pltpu.SemaphoreType.DMA,
        pltpu.SemaphoreType.DMA,
    ]
    return pl.kernel(
        functools.partial(_agg_body, scale),
        out_type=jax.ShapeDtypeStruct((NC, N, H), jnp.float32),
        mesh=_mesh,
        scratch_types=scratch,
        compiler_params=pltpu.CompilerParams(needs_layout_passes=False, use_tc_tiling_on_sc=False),
    )


_agg_weighted = _make_agg(True)
_agg_plain = _make_agg(False)


def _stats_body(pw_ref, pc_ref, pm_ref, d1_ref, d1s_ref, dc_ref, mk_ref):
    d1 = lax.rsqrt(jnp.sum(pw_ref[...], axis=(0, 1), keepdims=False)[None, :] + 1.0)
    dc = lax.rsqrt(jnp.sum(pc_ref[...], axis=(0, 1), keepdims=False)[None, :] + 1.0)
    mk = (jnp.sum(pm_ref[...], axis=(0, 1), keepdims=False)[None, :] > 0.0
          ).astype(jnp.float32)
    d1_ref[...] = jnp.transpose(d1)
    d1s_ref[...] = jnp.transpose(d1 * d1)
    dc_ref[...] = jnp.transpose(dc)
    mk_ref[...] = jnp.transpose(mk)


def _stats(pw, pc, pm):
    col = jax.ShapeDtypeStruct((N, 1), jnp.float32)
    return pl.pallas_call(
        _stats_body,
        out_shape=[col, col, col, col],
    )(pw, pc, pm)


def _xw1_body(x_ref, w_ref, out_ref):
    out_ref[...] = jnp.dot(x_ref[...], w_ref[...],
                           preferred_element_type=jnp.float32)


def _xw1(x, W1):
    blk = 2000
    return pl.pallas_call(
        _xw1_body,
        grid=(N // blk,),
        in_specs=[
            pl.BlockSpec((blk, x.shape[1]), lambda i: (i, 0)),
            pl.BlockSpec((x.shape[1], H), lambda i: (0, 0)),
        ],
        out_specs=pl.BlockSpec((blk, H), lambda i: (i, 0)),
        out_shape=jax.ShapeDtypeStruct((N, H), jnp.float32),
    )(x, W1)


def _layer_body(a_ref, y_ref, dva_ref, dvy_ref, dvl_ref, b_ref, w_ref, out_ref):
    h = jnp.maximum(
        dva_ref[...] * (a_ref[0] + a_ref[1]) + dvy_ref[...] * y_ref[...]
        + b_ref[...], 0.0
    )
    out_ref[...] = dvl_ref[...] * jnp.dot(
        h, w_ref[...], preferred_element_type=jnp.float32
    )


def _layer(a, y, dva, dvy, dv_l, b_prev, W_l):
    blk = 2000
    return pl.pallas_call(
        _layer_body,
        grid=(N // blk,),
        in_specs=[
            pl.BlockSpec((NC, blk, H), lambda i: (0, i, 0)),
            pl.BlockSpec((blk, H), lambda i: (i, 0)),
            pl.BlockSpec((blk, 1), lambda i: (i, 0)),
            pl.BlockSpec((blk, 1), lambda i: (i, 0)),
            pl.BlockSpec((blk, 1), lambda i: (i, 0)),
            pl.BlockSpec((1, H), lambda i: (0, 0)),
            pl.BlockSpec((H, H), lambda i: (0, 0)),
        ],
        out_specs=pl.BlockSpec((blk, H), lambda i: (i, 0)),
        out_shape=jax.ShapeDtypeStruct((N, H), jnp.float32),
    )(a, y, dva, dvy, dv_l, b_prev[None, :], W_l)


def _head_body(flags_ref, a_ref, y_ref, dvc_ref, mask_ref, b_ref,
               wp_ref, bp_ref, wv_ref, bv_ref, p_ref, v_ref):
    i = pl.program_id(0)
    act = flags_ref[i]
    dirty = jnp.where(i >= 2, flags_ref[jnp.maximum(i - 2, 0)], 1)

    @pl.when((act == 0) & (dirty != 0))
    def _zero():
        p_ref[...] = jnp.zeros_like(p_ref)
        v_ref[...] = jnp.zeros_like(v_ref)

    @pl.when(act != 0)
    def _compute():
        h = jnp.maximum(
            dvc_ref[...] * (a_ref[0] + a_ref[1] + y_ref[...]) + b_ref[...], 0.0
        )
        logits = jnp.dot(h, wp_ref[...], preferred_element_type=jnp.float32) + bp_ref[...]
        m = jnp.max(logits, axis=1, keepdims=True)
        e = jnp.exp(logits - m)
        s = jnp.sum(e, axis=1, keepdims=True)
        p_ref[...] = (e / s) * mask_ref[...]
        v_ref[...] = (
            jnp.dot(h, wv_ref[...], preferred_element_type=jnp.float32) + bv_ref[...]
        ) * mask_ref[...]


def _head(flags, a, y, dinvc, mask, b5, Wp, bp, Wv, bv):
    grid_spec = pltpu.PrefetchScalarGridSpec(
        num_scalar_prefetch=1,
        grid=(NB,),
        in_specs=[
            pl.BlockSpec((NC, BR, H), lambda i, f: (0, i, 0)),
            pl.BlockSpec((BR, H), lambda i, f: (i, 0)),
            pl.BlockSpec((BR, 1), lambda i, f: (i, 0)),
            pl.BlockSpec((BR, 1), lambda i, f: (i, 0)),
            pl.BlockSpec((1, H), lambda i, f: (0, 0)),
            pl.BlockSpec((H, O), lambda i, f: (0, 0)),
            pl.BlockSpec((1, O), lambda i, f: (0, 0)),
            pl.BlockSpec((H, O), lambda i, f: (0, 0)),
            pl.BlockSpec((1, O), lambda i, f: (0, 0)),
        ],
        out_specs=[
            pl.BlockSpec((BR, O), lambda i, f: (i, 0)),
            pl.BlockSpec((BR, O), lambda i, f: (i, 0)),
        ],
    )
    return pl.pallas_call(
        _head_body,
        grid_spec=grid_spec,
        out_shape=[
            jax.ShapeDtypeStruct((N, O), jnp.float32),
            jax.ShapeDtypeStruct((N, O), jnp.float32),
        ],
    )(flags, a, y, dinvc, mask, b5, Wp, bp, Wv, bv)


def kernel(x, edge_index, edge_weight, current_node,
           W1, b1, W2, b2, W3, b3, W4, b4, W5, b5, Wp, bp, Wv, bv):
    src = edge_index[0]
    dst = edge_index[1]
    cn = jnp.full((16,), current_node, jnp.int32)

    y = _xw1(x, W1)
    pw, pc, pm = _deg_call(dst, src, edge_weight, cn)
    dinv1, dinv1sq, dinvc, mask = _stats(pw, pc, pm)
    flags = (jnp.max(mask.reshape(NB, BR), axis=1) > 0.0).astype(jnp.int32)

    dst3 = dst.reshape(NW, EPT // CH, CH)
    a = _agg_weighted(y, src, dst3, edge_weight, dinv1[:, 0])

    dva, dvy, b_prev = dinv1, dinv1sq, b1
    for W_l, b_l in ((W2, b2), (W3, b3), (W4, b4), (W5, b5)):
        y = _layer(a, y, dva, dvy, dinvc, b_prev, W_l)
        a = _agg_plain(y, src, dst3)
        dva, dvy, b_prev = dinvc, dinvc, b_l

    p, v = _head(flags, a, y, dinvc, mask, b_prev[None, :],
                 Wp, bp[None, :], Wv, bv[None, :])
    return (p, v)


# speedup vs baseline: 1.0233x; 1.0233x over previous
import functools

import jax
import jax.numpy as jnp
from jax import lax
from jax.experimental import pallas as pl
from jax.experimental.pallas import tpu as pltpu
from jax.experimental.pallas import tpu_sc as plsc

N = 10000
E = 160000
H = 32
O = 10000

NC = 2
NS = 16
NW = NC * NS
EPT = E // NW
CH = 1000
NPT = N // NS
BR = 200
NB = N // BR

_mesh = plsc.VectorSubcoreMesh(core_axis_name="c", subcore_axis_name="s")

_Z16 = functools.partial(jnp.zeros, (16,), jnp.float32)


def _lane_bcast(vec16, u):
    return lax.gather(
        vec16,
        jnp.full((16, 1), u, jnp.int32),
        lax.GatherDimensionNumbers(
            offset_dims=(), collapsed_slice_dims=(0,), start_index_map=(0,)
        ),
        (1,),
        mode=lax.GatherScatterMode.PROMISE_IN_BOUNDS,
    )


def _vreg_loop(total, body16):
    nfull = total // 16

    def fbody(i, _):
        body16(i * 16, None)
        return 0

    lax.fori_loop(0, nfull, fbody, 0)
    rem = total - nfull * 16
    if rem:
        lane = lax.iota(jnp.int32, 16)
        body16(total - 16, lane >= (16 - rem))


def _deg_body(dst_h, src_h, ew_h, cn_h, outw_h, outc_h, outm_h,
              dst_v, src_v, ew_v, cn_v, accw, accc, acch):
    cid = lax.axis_index("c")
    sid = lax.axis_index("s")
    wid = cid * NS + sid
    z = _Z16()

    def zbody(i, _):
        accw[pl.ds(i * 16, 16)] = z
        accc[pl.ds(i * 16, 16)] = z
        acch[pl.ds(i * 16, 16)] = z
        return 0

    lax.fori_loop(0, N // 16, zbody, 0)

    base = wid * EPT
    pltpu.sync_copy(dst_h.at[pl.ds(base, EPT)], dst_v)
    pltpu.sync_copy(src_h.at[pl.ds(base, EPT)], src_v)
    pltpu.sync_copy(ew_h.at[pl.ds(base, EPT)], ew_v)
    pltpu.sync_copy(cn_h, cn_v)

    one = jnp.ones((16,), jnp.float32)
    cn16 = cn_v[pl.ds(0, 16)]

    def body16(off, mask):
        d16 = dst_v[pl.ds(off, 16)]
        hit16 = jnp.where(src_v[pl.ds(off, 16)] == cn16, 1.0, 0.0)
        plsc.addupdate_scatter(accw, [d16], ew_v[pl.ds(off, 16)], mask=mask)
        plsc.addupdate_scatter(accc, [d16], one, mask=mask)
        plsc.addupdate_scatter(acch, [d16], hit16, mask=mask)

    _vreg_loop(EPT, body16)

    pltpu.sync_copy(accw, outw_h.at[wid, 0])
    pltpu.sync_copy(accc, outc_h.at[wid, 0])
    pltpu.sync_copy(acch, outm_h.at[wid, 0])


_deg_call = pl.kernel(
    _deg_body,
    out_type=[
        jax.ShapeDtypeStruct((NW, 1, N), jnp.float32),
        jax.ShapeDtypeStruct((NW, 1, N), jnp.float32),
        jax.ShapeDtypeStruct((NW, 1, N), jnp.float32),
    ],
    mesh=_mesh,
    scratch_types=[
        pltpu.VMEM((EPT,), jnp.int32),
        pltpu.VMEM((EPT,), jnp.int32),
        pltpu.VMEM((EPT,), jnp.float32),
        pltpu.VMEM((16,), jnp.int32),
        pltpu.VMEM((N,), jnp.float32),
        pltpu.VMEM((N,), jnp.float32),
        pltpu.VMEM((N,), jnp.float32),
    ],
    compiler_params=pltpu.CompilerParams(needs_layout_passes=False, use_tc_tiling_on_sc=False),
)


def _agg_body(scale, *refs):
    if scale:
        (y_h, src_h, dst_h, w_h, out_h,
         src_v, dst_v, w_v, rows0, rows1, acc_sh, sem0, sem1) = refs
    else:
        (y_h, src_h, dst_h, out_h,
         src_v, dst_v, rows0, rows1, acc_sh, sem0, sem1) = refs
    cid = lax.axis_index("c")
    sid = lax.axis_index("s")
    wid = cid * NS + sid
    z = _Z16()

    def zbody(i, _):
        rows0[i, pl.ds(0, 16)] = z
        rows0[i, pl.ds(16, 16)] = z
        return 0

    lax.fori_loop(0, 640, zbody, 0)

    @pl.when(sid < NS - 1)
    def _zfill():
        pltpu.sync_copy(rows0.at[pl.ds(0, 624)], acc_sh.at[pl.ds(sid * 624, 624)])

    @pl.when(sid == NS - 1)
    def _zfill_last():
        pltpu.sync_copy(rows0.at[pl.ds(0, 640)], acc_sh.at[pl.ds(9360, 640)])

    plsc.subcore_barrier()

    base = wid * EPT
    pltpu.sync_copy(src_h.at[pl.ds(base, EPT)], src_v.at[pl.ds(0, EPT)])
    pltpu.sync_copy(dst_h.at[wid], dst_v)
    if scale:
        pltpu.sync_copy(w_h.at[pl.ds(base, EPT)], w_v.at[pl.ds(0, EPT)])

    rows = (rows0, rows1)
    sems = (sem0, sem1)
    NCH = EPT // CH
    descs = [None] * NCH
    descs[0] = pltpu.async_copy(y_h.at[src_v.at[pl.ds(0, CH)]], rows0, sems[0])
    for c in range(NCH):
        b = c % 2
        descs[c].wait()
        if c + 1 < NCH:
            descs[c + 1] = pltpu.async_copy(
                y_h.at[src_v.at[pl.ds((c + 1) * CH, CH)]],
                rows[(c + 1) % 2],
                sems[(c + 1) % 2],
            )
        if scale:
            rv = rows[b]

            def sbody(i, _):
                w16 = w_v[pl.ds(c * CH + i * 8, 16)]
                for u in range(8):
                    e = i * 8 + u
                    wb = _lane_bcast(w16, u)
                    rv[e, pl.ds(0, 16)] = rv[e, pl.ds(0, 16)] * wb
                    rv[e, pl.ds(16, 16)] = rv[e, pl.ds(16, 16)] * wb
                return 0

            lax.fori_loop(0, CH // 8, sbody, 0)
        pltpu.sync_copy(rows[b], acc_sh.at[dst_v.at[c]], add=True)

    plsc.subcore_barrier()

    @pl.when(sid < NS - 1)
    def _wout():
        pltpu.sync_copy(
            acc_sh.at[pl.ds(sid * 624, 624)],
            out_h.at[cid, pl.ds(sid * 624, 624)],
        )

    @pl.when(sid == NS - 1)
    def _wout_last():
        pltpu.sync_copy(
            acc_sh.at[pl.ds(9360, 640)],
            out_h.at[cid, pl.ds(9360, 640)],
        )


def _make_agg(scale):
    if scale:
        scratch = [
            pltpu.VMEM((EPT,), jnp.int32),
            pltpu.VMEM((EPT // CH, CH), jnp.int32),
            pltpu.VMEM((EPT + 16,), jnp.float32),
        ]
    else:
        scratch = [
            pltpu.VMEM((EPT,), jnp.int32),
            pltpu.VMEM((EPT // CH, CH), jnp.int32),
        ]
    scratch += [
        pltpu.VMEM((CH, H), jnp.float32),
        pltpu.VMEM((CH, H), jnp.float32),
        pltpu.VMEM_SHARED((N, H), jnp.float32),
        pltpu.SemaphoreType.DMA,
        pltpu.SemaphoreType.DMA,
    ]
    return pl.kernel(
        functools.partial(_agg_body, scale),
        out_type=jax.ShapeDtypeStruct((NC, N, H), jnp.float32),
        mesh=_mesh,
        scratch_types=scratch,
        compiler_params=pltpu.CompilerParams(needs_layout_passes=False, use_tc_tiling_on_sc=False),
    )


_agg_weighted = _make_agg(True)
_agg_plain = _make_agg(False)


def _stats_body(pw_ref, pc_ref, pm_ref, d1_ref, dc_ref, mk_ref):
    d1 = lax.rsqrt(jnp.sum(pw_ref[...], axis=(0, 1), keepdims=False)[None, :] + 1.0)
    dc = lax.rsqrt(jnp.sum(pc_ref[...], axis=(0, 1), keepdims=False)[None, :] + 1.0)
    mk = (jnp.sum(pm_ref[...], axis=(0, 1), keepdims=False)[None, :] > 0.0
          ).astype(jnp.float32)
    d1_ref[...] = jnp.transpose(d1)
    dc_ref[...] = jnp.transpose(dc)
    mk_ref[...] = jnp.transpose(mk)


def _stats(pw, pc, pm):
    col = jax.ShapeDtypeStruct((N, 1), jnp.float32)
    return pl.pallas_call(
        _stats_body,
        out_shape=[col, col, col],
    )(pw, pc, pm)


def _y1_body(x_ref, w_ref, dinv_ref, out_ref):
    xw = jnp.dot(x_ref[...], w_ref[...], preferred_element_type=jnp.float32)
    out_ref[...] = dinv_ref[...] * xw


def _y1(x, W1, dinv1):
    blk = 2000
    return pl.pallas_call(
        _y1_body,
        grid=(N // blk,),
        in_specs=[
            pl.BlockSpec((blk, x.shape[1]), lambda i: (i, 0)),
            pl.BlockSpec((x.shape[1], H), lambda i: (0, 0)),
            pl.BlockSpec((blk, 1), lambda i: (i, 0)),
        ],
        out_specs=pl.BlockSpec((blk, H), lambda i: (i, 0)),
        out_shape=jax.ShapeDtypeStruct((N, H), jnp.float32),
    )(x, W1, dinv1)


def _layer_body(a_ref, y_ref, dvp_ref, dvl_ref, b_ref, w_ref, out_ref):
    h = jnp.maximum(
        dvp_ref[...] * (a_ref[0] + a_ref[1] + y_ref[...]) + b_ref[...], 0.0
    )
    out_ref[...] = dvl_ref[...] * jnp.dot(
        h, w_ref[...], preferred_element_type=jnp.float32
    )


def _layer(a, y, dv_prev, dv_l, b_prev, W_l):
    blk = 2000
    return pl.pallas_call(
        _layer_body,
        grid=(N // blk,),
        in_specs=[
            pl.BlockSpec((NC, blk, H), lambda i: (0, i, 0)),
            pl.BlockSpec((blk, H), lambda i: (i, 0)),
            pl.BlockSpec((blk, 1), lambda i: (i, 0)),
            pl.BlockSpec((blk, 1), lambda i: (i, 0)),
            pl.BlockSpec((1, H), lambda i: (0, 0)),
            pl.BlockSpec((H, H), lambda i: (0, 0)),
        ],
        out_specs=pl.BlockSpec((blk, H), lambda i: (i, 0)),
        out_shape=jax.ShapeDtypeStruct((N, H), jnp.float32),
    )(a, y, dv_prev, dv_l, b_prev[None, :], W_l)


def _head_body(flags_ref, a_ref, y_ref, dvc_ref, mask_ref, b_ref,
               wp_ref, bp_ref, wv_ref, bv_ref, p_ref, v_ref):
    i = pl.program_id(0)
    act = flags_ref[i]
    dirty = jnp.where(i >= 2, flags_ref[jnp.maximum(i - 2, 0)], 1)

    @pl.when((act == 0) & (dirty != 0))
    def _zero():
        p_ref[...] = jnp.zeros_like(p_ref)
        v_ref[...] = jnp.zeros_like(v_ref)

    @pl.when(act != 0)
    def _compute():
        h = jnp.maximum(
            dvc_ref[...] * (a_ref[0] + a_ref[1] + y_ref[...]) + b_ref[...], 0.0
        )
        logits = jnp.dot(h, wp_ref[...], preferred_element_type=jnp.float32) + bp_ref[...]
        m = jnp.max(logits, axis=1, keepdims=True)
        e = jnp.exp(logits - m)
        s = jnp.sum(e, axis=1, keepdims=True)
        p_ref[...] = (e / s) * mask_ref[...]
        v_ref[...] = (
            jnp.dot(h, wv_ref[...], preferred_element_type=jnp.float32) + bv_ref[...]
        ) * mask_ref[...]


def _head(flags, a, y, dinvc, mask, b5, Wp, bp, Wv, bv):
    grid_spec = pltpu.PrefetchScalarGridSpec(
        num_scalar_prefetch=1,
        grid=(NB,),
        in_specs=[
            pl.BlockSpec((NC, BR, H), lambda i, f: (0, i, 0)),
            pl.BlockSpec((BR, H), lambda i, f: (i, 0)),
            pl.BlockSpec((BR, 1), lambda i, f: (i, 0)),
            pl.BlockSpec((BR, 1), lambda i, f: (i, 0)),
            pl.BlockSpec((1, H), lambda i, f: (0, 0)),
            pl.BlockSpec((H, O), lambda i, f: (0, 0)),
            pl.BlockSpec((1, O), lambda i, f: (0, 0)),
            pl.BlockSpec((H, O), lambda i, f: (0, 0)),
            pl.BlockSpec((1, O), lambda i, f: (0, 0)),
        ],
        out_specs=[
            pl.BlockSpec((BR, O), lambda i, f: (i, 0)),
            pl.BlockSpec((BR, O), lambda i, f: (i, 0)),
        ],
    )
    return pl.pallas_call(
        _head_body,
        grid_spec=grid_spec,
        out_shape=[
            jax.ShapeDtypeStruct((N, O), jnp.float32),
            jax.ShapeDtypeStruct((N, O), jnp.float32),
        ],
    )(flags, a, y, dinvc, mask, b5, Wp, bp, Wv, bv)


def kernel(x, edge_index, edge_weight, current_node,
           W1, b1, W2, b2, W3, b3, W4, b4, W5, b5, Wp, bp, Wv, bv):
    src = edge_index[0]
    dst = edge_index[1]
    cn = jnp.full((16,), current_node, jnp.int32)

    pw, pc, pm = _deg_call(dst, src, edge_weight, cn)
    dinv1, dinvc, mask = _stats(pw, pc, pm)
    flags = (jnp.max(mask.reshape(NB, BR), axis=1) > 0.0).astype(jnp.int32)

    dst3 = dst.reshape(NW, EPT // CH, CH)
    y = _y1(x, W1, dinv1)
    a = _agg_weighted(y, src, dst3, edge_weight)

    dv_prev, b_prev = dinv1, b1
    for W_l, b_l in ((W2, b2), (W3, b3), (W4, b4), (W5, b5)):
        y = _layer(a, y, dv_prev, dinvc, b_prev, W_l)
        a = _agg_plain(y, src, dst3)
        dv_prev, b_prev = dinvc, b_l

    p, v = _head(flags, a, y, dinvc, mask, b_prev[None, :],
                 Wp, bp[None, :], Wv, bv[None, :])
    return (p, v)
